# baseline (speedup 1.0000x reference)
import numpy as np

import jax
import jax.numpy as jnp
from jax.experimental import pallas as pl
from jax.experimental.pallas import tpu as pltpu

KSIZE = 5
GROUP = 16


def _pool_select(n_rows, wp_in, h_out, w_out, wp_out, pad, k_pad):
    s = np.zeros((n_rows, k_pad), np.float32)
    for ho in range(h_out):
        for wo in range(w_out):
            s[(ho + pad) * wp_out + (wo + pad), 2 * ho * wp_in + 2 * wo] = 1.0
    return jnp.asarray(s, jnp.bfloat16)


def _tower_kernel(xp_ref, w1_ref, w2_ref, w3_ref, b1_ref, b2_ref, b3_ref,
                  s1_ref, s2_ref, s3_ref, o_ref,
                  x1_ref, acc1_ref, m1_ref, x2_ref, x2im_refs, m2_refs,
                  x3_ref, x3im_refs, m3_refs):
    for t in range(KSIZE * KSIZE):
        kh, kw = divmod(t, KSIZE)
        off = kh * 36 + kw
        x1_ref[:, 64 * t:64 * (t + 1)] = xp_ref[0, pl.ds(off, 1152), :]
    acc1_ref[...] = jnp.dot(x1_ref[...], w1_ref[...],
                            preferred_element_type=jnp.float32)

    m1 = jnp.maximum(
        jnp.maximum(acc1_ref[pl.ds(0, 1115), :], acc1_ref[pl.ds(1, 1115), :]),
        jnp.maximum(acc1_ref[pl.ds(36, 1115), :], acc1_ref[pl.ds(37, 1115), :]))
    m1_ref[pl.ds(0, 1115), :] = (m1 + b1_ref[...]).astype(jnp.bfloat16)
    m1_ref[pl.ds(1115, 5), :] = jnp.zeros((5, 512), jnp.bfloat16)
    x2_ref[...] = jnp.dot(s1_ref[...], m1_ref[...],
                          preferred_element_type=jnp.float32
                          ).astype(jnp.bfloat16)

    for g in range(2):
        m2_ref = m2_refs[g]
        x2im_ref = x2im_refs[g]
        for t in range(KSIZE * KSIZE):
            kh, kw = divmod(t, KSIZE)
            off = kh * 20 + kw
            x2im_ref[:, 256 * t:256 * (t + 1)] = \
                x2_ref[pl.ds(off, 320), 256 * g:256 * (g + 1)]
        acc = jnp.dot(x2im_ref[...], w2_ref[...],
                      preferred_element_type=jnp.float32)
        m2 = jnp.maximum(jnp.maximum(acc[0:299], acc[1:300]),
                         jnp.maximum(acc[20:319], acc[21:320]))
        m2_ref[pl.ds(0, 299), :] = (m2 + b2_ref[...]).astype(jnp.bfloat16)
        m2_ref[pl.ds(299, 21), :] = jnp.zeros((21, 256), jnp.bfloat16)
        x3_ref[:, 256 * g:256 * (g + 1)] = jnp.dot(
            s2_ref[...], m2_ref[...],
            preferred_element_type=jnp.float32).astype(jnp.bfloat16)

    for g in range(4):
        m3_ref = m3_refs[g]
        x3im_ref = x3im_refs[g % 2]
        for t in range(KSIZE * KSIZE):
            kh, kw = divmod(t, KSIZE)
            off = kh * 12 + kw
            x3im_ref[:, 128 * t:128 * (t + 1)] = \
                x3_ref[pl.ds(off, 96), 128 * g:128 * (g + 1)]
        acc = jnp.dot(x3im_ref[...], w3_ref[...],
                      preferred_element_type=jnp.float32)
        m3 = jnp.maximum(jnp.maximum(acc[0:83], acc[1:84]),
                         jnp.maximum(acc[12:95], acc[13:96]))
        m3_ref[pl.ds(0, 83), :] = (m3 + b3_ref[...]).astype(jnp.bfloat16)
        m3_ref[pl.ds(83, 5), :] = jnp.zeros((5, 256), jnp.bfloat16)
        o_ref[0, :, 256 * g:256 * (g + 1)] = jnp.dot(
            s3_ref[...], m3_ref[...],
            preferred_element_type=jnp.float32).astype(jnp.bfloat16)


def _mlp_kernel(x_ref, w1_ref, b1_ref, w2_ref, b2_ref, o_ref):
    h = jnp.dot(x_ref[...], w1_ref[...],
                preferred_element_type=jnp.float32) + b1_ref[...]
    o_ref[...] = jnp.dot(h, w2_ref[...],
                         preferred_element_type=jnp.float32) + b2_ref[...]


@jax.jit
def kernel(w1, b1, w2, b2, w3, b3, fw1, fb1, fw2, fb2, x):
    B = x.shape[0]
    S = B // GROUP

    xb = x.astype(jnp.bfloat16).reshape(S, GROUP, 3, 32, 32)
    xb = jnp.transpose(xb, (0, 3, 4, 1, 2))
    xb = jnp.pad(xb, ((0, 0), (2, 3), (2, 2), (0, 0), (0, 1)))
    xp = xb.reshape(S, 37 * 36, GROUP * 4)

    eye16 = jnp.eye(16, dtype=w1.dtype)
    eye8 = jnp.eye(8, dtype=w1.dtype)
    eye4 = jnp.eye(4, dtype=w1.dtype)
    w1bd = jnp.einsum('ab,tkc->takbc', eye16, w1[:, :4, :32])
    w1im = w1bd.reshape(25 * 64, 512).astype(jnp.bfloat16)
    w2bd = jnp.einsum('ab,tkc->takbc', eye8, w2[:, :32, :32])
    w2bd = w2bd.reshape(25 * 256, 256).astype(jnp.bfloat16)
    w3bd = jnp.einsum('ab,tkc->takbc', eye4, w3[:, :32, :64])
    w3bd = w3bd.reshape(25 * 128, 256).astype(jnp.bfloat16)
    b1t = jnp.tile(b1[:, :32], (1, 16))
    b2t = jnp.tile(b2[:, :32], (1, 8))
    b3t = jnp.tile(b3[:, :64], (1, 4))

    s1 = _pool_select(420, 36, 16, 16, 20, 2, 1120)
    s2 = _pool_select(156, 20, 8, 8, 12, 2, 320)
    s3 = _pool_select(16, 12, 4, 4, 4, 0, 88)

    pooled = pl.pallas_call(
        _tower_kernel,
        out_shape=jax.ShapeDtypeStruct((S, 16, 1024), jnp.bfloat16),
        grid=(S,),
        in_specs=[
            pl.BlockSpec((1, 1332, 64), lambda s: (s, 0, 0)),
            pl.BlockSpec((25 * 64, 512), lambda s: (0, 0)),
            pl.BlockSpec((25 * 256, 256), lambda s: (0, 0)),
            pl.BlockSpec((25 * 128, 256), lambda s: (0, 0)),
            pl.BlockSpec((1, 512), lambda s: (0, 0)),
            pl.BlockSpec((1, 256), lambda s: (0, 0)),
            pl.BlockSpec((1, 256), lambda s: (0, 0)),
            pl.BlockSpec((420, 1120), lambda s: (0, 0)),
            pl.BlockSpec((156, 320), lambda s: (0, 0)),
            pl.BlockSpec((16, 88), lambda s: (0, 0)),
        ],
        out_specs=pl.BlockSpec((1, 16, 1024), lambda s: (s, 0, 0)),
        scratch_shapes=[
            pltpu.VMEM((1152, 1600), jnp.bfloat16),
            pltpu.VMEM((1152, 512), jnp.float32),
            pltpu.VMEM((1120, 512), jnp.bfloat16),
            pltpu.VMEM((420, 512), jnp.bfloat16),
            tuple(pltpu.VMEM((320, 6400), jnp.bfloat16) for _ in range(2)),
            tuple(pltpu.VMEM((320, 256), jnp.bfloat16) for _ in range(2)),
            pltpu.VMEM((156, 512), jnp.bfloat16),
            tuple(pltpu.VMEM((96, 3200), jnp.bfloat16) for _ in range(2)),
            tuple(pltpu.VMEM((88, 256), jnp.bfloat16) for _ in range(4)),
        ],
        compiler_params=pltpu.CompilerParams(
            dimension_semantics=("parallel",)),
    )(xp, w1im, w2bd, w3bd, b1t, b2t, b3t, s1, s2, s3)

    xf = pooled.reshape(S, 16, 4, 4, 64).transpose(0, 2, 3, 1, 4)
    xf = xf.reshape(B, 1024)

    fw1r = fw1.reshape(4, 4, 128, 64)[:, :, :64, :]
    fw1r = fw1r.reshape(1024, 64).astype(jnp.bfloat16)

    MB = 1024 if B % 1024 == 0 else B
    return pl.pallas_call(
        _mlp_kernel,
        out_shape=jax.ShapeDtypeStruct((B, 10), jnp.float32),
        grid=(B // MB,),
        in_specs=[
            pl.BlockSpec((MB, 1024), lambda s: (s, 0)),
            pl.BlockSpec((1024, 64), lambda s: (0, 0)),
            pl.BlockSpec((1, 64), lambda s: (0, 0)),
            pl.BlockSpec((64, 10), lambda s: (0, 0)),
            pl.BlockSpec((1, 10), lambda s: (0, 0)),
        ],
        out_specs=pl.BlockSpec((MB, 10), lambda s: (s, 0)),
        compiler_params=pltpu.CompilerParams(
            dimension_semantics=("parallel",)),
    )(xf, fw1r, fb1, fw2, fb2)


# speedup vs baseline: 3.5834x; 3.5834x over previous
import numpy as np

import jax
import jax.numpy as jnp
from jax.experimental import pallas as pl
from jax.experimental.pallas import tpu as pltpu

KSIZE = 5
GROUP = 16


def _pool_select(n_rows, wp_in, h_out, w_out, wp_out, pad, k_pad):
    s = np.zeros((n_rows, k_pad), np.float32)
    for ho in range(h_out):
        for wo in range(w_out):
            s[(ho + pad) * wp_out + (wo + pad), 2 * ho * wp_in + 2 * wo] = 1.0
    return jnp.asarray(s, jnp.bfloat16)


def _pad_permute():
    p = np.zeros((1024, 1408), np.float32)
    for h in range(32):
        for w in range(32):
            p[h * 32 + w, (h + 2) * 36 + (w + 2)] = 1.0
    return jnp.asarray(p, jnp.bfloat16)


def _tower_kernel(xf_ref, p_ref, w1_ref, w2_ref, w3_ref,
                  b1_ref, b2_ref, b3_ref,
                  s1_ref, s2_ref, s3_ref, o_ref,
                  xpt_ref, x1_ref, acc1_ref, m1_ref, x2_ref, x2im_refs,
                  m2_refs, x3_ref, x3im_refs, m3_refs):
    xpt_ref[...] = jnp.dot(xf_ref[0], p_ref[...],
                           preferred_element_type=jnp.float32
                           ).astype(jnp.bfloat16)
    for t in range(KSIZE * KSIZE):
        kh, kw = divmod(t, KSIZE)
        off = kh * 36 + kw
        x1_ref[48 * t:48 * (t + 1), :] = xpt_ref[:, pl.ds(off, 1152)]
    acc1_ref[...] = jax.lax.dot_general(
        x1_ref[...], w1_ref[...], (((0,), (0,)), ((), ())),
        preferred_element_type=jnp.float32)

    m1 = jnp.maximum(
        jnp.maximum(acc1_ref[pl.ds(0, 1115), :], acc1_ref[pl.ds(1, 1115), :]),
        jnp.maximum(acc1_ref[pl.ds(36, 1115), :], acc1_ref[pl.ds(37, 1115), :]))
    m1_ref[pl.ds(0, 1115), :] = (m1 + b1_ref[...]).astype(jnp.bfloat16)
    m1_ref[pl.ds(1115, 5), :] = jnp.zeros((5, 512), jnp.bfloat16)
    x2_ref[...] = jnp.dot(s1_ref[...], m1_ref[...],
                          preferred_element_type=jnp.float32
                          ).astype(jnp.bfloat16)

    for g in range(2):
        m2_ref = m2_refs[g]
        x2im_ref = x2im_refs[g]
        for t in range(KSIZE * KSIZE):
            kh, kw = divmod(t, KSIZE)
            off = kh * 20 + kw
            x2im_ref[:, 256 * t:256 * (t + 1)] = \
                x2_ref[pl.ds(off, 320), 256 * g:256 * (g + 1)]
        acc = jnp.dot(x2im_ref[...], w2_ref[...],
                      preferred_element_type=jnp.float32)
        m2 = jnp.maximum(jnp.maximum(acc[0:299], acc[1:300]),
                         jnp.maximum(acc[20:319], acc[21:320]))
        m2_ref[pl.ds(0, 299), :] = (m2 + b2_ref[...]).astype(jnp.bfloat16)
        m2_ref[pl.ds(299, 21), :] = jnp.zeros((21, 256), jnp.bfloat16)
        x3_ref[:, 256 * g:256 * (g + 1)] = jnp.dot(
            s2_ref[...], m2_ref[...],
            preferred_element_type=jnp.float32).astype(jnp.bfloat16)

    for g in range(4):
        m3_ref = m3_refs[g]
        x3im_ref = x3im_refs[g % 2]
        for t in range(KSIZE * KSIZE):
            kh, kw = divmod(t, KSIZE)
            off = kh * 12 + kw
            x3im_ref[:, 128 * t:128 * (t + 1)] = \
                x3_ref[pl.ds(off, 96), 128 * g:128 * (g + 1)]
        acc = jnp.dot(x3im_ref[...], w3_ref[...],
                      preferred_element_type=jnp.float32)
        m3 = jnp.maximum(jnp.maximum(acc[0:83], acc[1:84]),
                         jnp.maximum(acc[12:95], acc[13:96]))
        m3_ref[pl.ds(0, 83), :] = (m3 + b3_ref[...]).astype(jnp.bfloat16)
        m3_ref[pl.ds(83, 5), :] = jnp.zeros((5, 256), jnp.bfloat16)
        o_ref[0, :, 256 * g:256 * (g + 1)] = jnp.dot(
            s3_ref[...], m3_ref[...],
            preferred_element_type=jnp.float32).astype(jnp.bfloat16)


def _mlp_kernel(x_ref, w1_ref, b1_ref, w2_ref, b2_ref, o_ref):
    h = jnp.dot(x_ref[...], w1_ref[...],
                preferred_element_type=jnp.float32) + b1_ref[...]
    o_ref[...] = jnp.dot(h, w2_ref[...],
                         preferred_element_type=jnp.float32) + b2_ref[...]


@jax.jit
def kernel(w1, b1, w2, b2, w3, b3, fw1, fb1, fw2, fb2, x):
    B = x.shape[0]
    S = B // GROUP

    xf = x.astype(jnp.bfloat16).reshape(S, GROUP * 3, 1024)

    eye16 = jnp.eye(16, dtype=w1.dtype)
    eye8 = jnp.eye(8, dtype=w1.dtype)
    eye4 = jnp.eye(4, dtype=w1.dtype)
    w1bd = jnp.einsum('ab,tkc->takbc', eye16, w1[:, :3, :32])
    w1im = w1bd.reshape(25 * 48, 512).astype(jnp.bfloat16)
    w2bd = jnp.einsum('ab,tkc->takbc', eye8, w2[:, :32, :32])
    w2bd = w2bd.reshape(25 * 256, 256).astype(jnp.bfloat16)
    w3bd = jnp.einsum('ab,tkc->takbc', eye4, w3[:, :32, :64])
    w3bd = w3bd.reshape(25 * 128, 256).astype(jnp.bfloat16)
    b1t = jnp.tile(b1[:, :32], (1, 16))
    b2t = jnp.tile(b2[:, :32], (1, 8))
    b3t = jnp.tile(b3[:, :64], (1, 4))

    s1 = _pool_select(420, 36, 16, 16, 20, 2, 1120)
    s2 = _pool_select(156, 20, 8, 8, 12, 2, 320)
    s3 = _pool_select(16, 12, 4, 4, 4, 0, 88)

    pooled = pl.pallas_call(
        _tower_kernel,
        out_shape=jax.ShapeDtypeStruct((S, 16, 1024), jnp.bfloat16),
        grid=(S,),
        in_specs=[
            pl.BlockSpec((1, GROUP * 3, 1024), lambda s: (s, 0, 0)),
            pl.BlockSpec((1024, 1408), lambda s: (0, 0)),
            pl.BlockSpec((25 * 48, 512), lambda s: (0, 0)),
            pl.BlockSpec((25 * 256, 256), lambda s: (0, 0)),
            pl.BlockSpec((25 * 128, 256), lambda s: (0, 0)),
            pl.BlockSpec((1, 512), lambda s: (0, 0)),
            pl.BlockSpec((1, 256), lambda s: (0, 0)),
            pl.BlockSpec((1, 256), lambda s: (0, 0)),
            pl.BlockSpec((420, 1120), lambda s: (0, 0)),
            pl.BlockSpec((156, 320), lambda s: (0, 0)),
            pl.BlockSpec((16, 88), lambda s: (0, 0)),
        ],
        out_specs=pl.BlockSpec((1, 16, 1024), lambda s: (s, 0, 0)),
        scratch_shapes=[
            pltpu.VMEM((48, 1408), jnp.bfloat16),
            pltpu.VMEM((1200, 1152), jnp.bfloat16),
            pltpu.VMEM((1152, 512), jnp.float32),
            pltpu.VMEM((1120, 512), jnp.bfloat16),
            pltpu.VMEM((420, 512), jnp.bfloat16),
            tuple(pltpu.VMEM((320, 6400), jnp.bfloat16) for _ in range(2)),
            tuple(pltpu.VMEM((320, 256), jnp.bfloat16) for _ in range(2)),
            pltpu.VMEM((156, 512), jnp.bfloat16),
            tuple(pltpu.VMEM((96, 3200), jnp.bfloat16) for _ in range(2)),
            tuple(pltpu.VMEM((88, 256), jnp.bfloat16) for _ in range(4)),
        ],
        compiler_params=pltpu.CompilerParams(
            dimension_semantics=("parallel",)),
    )(xf, _pad_permute(), w1im, w2bd, w3bd, b1t, b2t, b3t, s1, s2, s3)

    xf = pooled.reshape(S, 16, 4, 4, 64).transpose(0, 2, 3, 1, 4)
    xf = xf.reshape(B, 1024)

    fw1r = fw1.reshape(4, 4, 128, 64)[:, :, :64, :]
    fw1r = fw1r.reshape(1024, 64).astype(jnp.bfloat16)

    MB = 1024 if B % 1024 == 0 else B
    return pl.pallas_call(
        _mlp_kernel,
        out_shape=jax.ShapeDtypeStruct((B, 10), jnp.float32),
        grid=(B // MB,),
        in_specs=[
            pl.BlockSpec((MB, 1024), lambda s: (s, 0)),
            pl.BlockSpec((1024, 64), lambda s: (0, 0)),
            pl.BlockSpec((1, 64), lambda s: (0, 0)),
            pl.BlockSpec((64, 10), lambda s: (0, 0)),
            pl.BlockSpec((1, 10), lambda s: (0, 0)),
        ],
        out_specs=pl.BlockSpec((MB, 10), lambda s: (s, 0)),
        compiler_params=pltpu.CompilerParams(
            dimension_semantics=("parallel",)),
    )(xf, fw1r, fb1, fw2, fb2)


# speedup vs baseline: 4.0782x; 1.1381x over previous
import numpy as np

import jax
import jax.numpy as jnp
from jax.experimental import pallas as pl
from jax.experimental.pallas import tpu as pltpu

KSIZE = 5
GROUP = 16


def _pool_select(n_rows, wp_in, h_out, w_out, wp_out, pad, k_pad):
    s = np.zeros((n_rows, k_pad), np.float32)
    for ho in range(h_out):
        for wo in range(w_out):
            s[(ho + pad) * wp_out + (wo + pad), 2 * ho * wp_in + 2 * wo] = 1.0
    return jnp.asarray(s, jnp.bfloat16)


def _pad_permute():
    p = np.zeros((1024, 1408), np.float32)
    for h in range(32):
        for w in range(32):
            p[h * 32 + w, (h + 2) * 36 + (w + 2)] = 1.0
    return jnp.asarray(p, jnp.bfloat16)


def _tower_kernel(xf_ref, p_ref, w1_ref, w2_ref, w3_ref,
                  b1_ref, b2_ref, b3_ref,
                  s1_ref, s2_ref, s3_ref, o_ref,
                  xpt_ref, x1_ref, acc1_ref, m1_ref, x2_ref,
                  m2_refs, x3_ref, m3_refs):
    xpt_ref[...] = jnp.dot(xf_ref[0], p_ref[...],
                           preferred_element_type=jnp.float32
                           ).astype(jnp.bfloat16)
    for t in range(KSIZE * KSIZE):
        kh, kw = divmod(t, KSIZE)
        off = kh * 36 + kw
        x1_ref[48 * t:48 * (t + 1), :] = xpt_ref[:, pl.ds(off, 1152)]
    acc1_ref[...] = jax.lax.dot_general(
        x1_ref[...], w1_ref[...], (((0,), (0,)), ((), ())),
        preferred_element_type=jnp.float32)

    m1 = jnp.maximum(
        jnp.maximum(acc1_ref[pl.ds(0, 1115), :], acc1_ref[pl.ds(1, 1115), :]),
        jnp.maximum(acc1_ref[pl.ds(36, 1115), :], acc1_ref[pl.ds(37, 1115), :]))
    m1_ref[pl.ds(0, 1115), :] = (m1 + b1_ref[...]).astype(jnp.bfloat16)
    m1_ref[pl.ds(1115, 5), :] = jnp.zeros((5, 512), jnp.bfloat16)
    x2_ref[...] = jnp.dot(s1_ref[...], m1_ref[...],
                          preferred_element_type=jnp.float32
                          ).astype(jnp.bfloat16)

    for g in range(2):
        m2_ref = m2_refs[g]
        for si in range(2):
            base = 160 * si
            acc = None
            for t in range(KSIZE * KSIZE):
                kh, kw = divmod(t, KSIZE)
                off = base + kh * 20 + kw
                xs = x2_ref[pl.ds(off, 160), 256 * g:256 * (g + 1)]
                contrib = jnp.dot(xs, w2_ref[t, :, :],
                                  preferred_element_type=jnp.float32)
                acc = contrib if acc is None else acc + contrib
            m2 = jnp.maximum(jnp.maximum(acc[0:139], acc[1:140]),
                             jnp.maximum(acc[20:159], acc[21:160]))
            m2_ref[pl.ds(base, 139), :] = (m2 + b2_ref[...]).astype(jnp.bfloat16)
            m2_ref[pl.ds(base + 139, 21), :] = jnp.zeros((21, 256),
                                                         jnp.bfloat16)
        x3_ref[:, 256 * g:256 * (g + 1)] = jnp.dot(
            s2_ref[...], m2_ref[...],
            preferred_element_type=jnp.float32).astype(jnp.bfloat16)

    for g in range(4):
        m3_ref = m3_refs[g]
        acc = None
        for t in range(KSIZE * KSIZE):
            kh, kw = divmod(t, KSIZE)
            off = kh * 12 + kw
            xs = x3_ref[pl.ds(off, 96), 128 * g:128 * (g + 1)]
            contrib = jnp.dot(xs, w3_ref[t, :, :],
                              preferred_element_type=jnp.float32)
            acc = contrib if acc is None else acc + contrib
        m3 = jnp.maximum(jnp.maximum(acc[0:83], acc[1:84]),
                         jnp.maximum(acc[12:95], acc[13:96]))
        m3_ref[pl.ds(0, 83), :] = (m3 + b3_ref[...]).astype(jnp.bfloat16)
        m3_ref[pl.ds(83, 5), :] = jnp.zeros((5, 256), jnp.bfloat16)
        o_ref[0, :, 256 * g:256 * (g + 1)] = jnp.dot(
            s3_ref[...], m3_ref[...],
            preferred_element_type=jnp.float32).astype(jnp.bfloat16)


def _mlp_kernel(x_ref, w1_ref, b1_ref, w2_ref, b2_ref, o_ref):
    h = jnp.dot(x_ref[...], w1_ref[...],
                preferred_element_type=jnp.float32) + b1_ref[...]
    o_ref[...] = jnp.dot(h, w2_ref[...],
                         preferred_element_type=jnp.float32) + b2_ref[...]


@jax.jit
def kernel(w1, b1, w2, b2, w3, b3, fw1, fb1, fw2, fb2, x):
    B = x.shape[0]
    S = B // GROUP

    xf = x.astype(jnp.bfloat16).reshape(S, GROUP * 3, 1024)

    eye16 = jnp.eye(16, dtype=w1.dtype)
    eye8 = jnp.eye(8, dtype=w1.dtype)
    eye4 = jnp.eye(4, dtype=w1.dtype)
    w1bd = jnp.einsum('ab,tkc->takbc', eye16, w1[:, :3, :32])
    w1im = w1bd.reshape(25 * 48, 512).astype(jnp.bfloat16)
    w2bd = jnp.einsum('ab,tkc->takbc', eye8, w2[:, :32, :32])
    w2bd = w2bd.reshape(25, 256, 256).astype(jnp.bfloat16)
    w3bd = jnp.einsum('ab,tkc->takbc', eye4, w3[:, :32, :64])
    w3bd = w3bd.reshape(25, 128, 256).astype(jnp.bfloat16)
    b1t = jnp.tile(b1[:, :32], (1, 16))
    b2t = jnp.tile(b2[:, :32], (1, 8))
    b3t = jnp.tile(b3[:, :64], (1, 4))

    s1 = _pool_select(420, 36, 16, 16, 20, 2, 1120)
    s2 = _pool_select(156, 20, 8, 8, 12, 2, 320)
    s3 = _pool_select(16, 12, 4, 4, 4, 0, 88)

    pooled = pl.pallas_call(
        _tower_kernel,
        out_shape=jax.ShapeDtypeStruct((S, 16, 1024), jnp.bfloat16),
        grid=(S,),
        in_specs=[
            pl.BlockSpec((1, GROUP * 3, 1024), lambda s: (s, 0, 0)),
            pl.BlockSpec((1024, 1408), lambda s: (0, 0)),
            pl.BlockSpec((25 * 48, 512), lambda s: (0, 0)),
            pl.BlockSpec((25, 256, 256), lambda s: (0, 0, 0)),
            pl.BlockSpec((25, 128, 256), lambda s: (0, 0, 0)),
            pl.BlockSpec((1, 512), lambda s: (0, 0)),
            pl.BlockSpec((1, 256), lambda s: (0, 0)),
            pl.BlockSpec((1, 256), lambda s: (0, 0)),
            pl.BlockSpec((420, 1120), lambda s: (0, 0)),
            pl.BlockSpec((156, 320), lambda s: (0, 0)),
            pl.BlockSpec((16, 88), lambda s: (0, 0)),
        ],
        out_specs=pl.BlockSpec((1, 16, 1024), lambda s: (s, 0, 0)),
        scratch_shapes=[
            pltpu.VMEM((48, 1408), jnp.bfloat16),
            pltpu.VMEM((1200, 1152), jnp.bfloat16),
            pltpu.VMEM((1152, 512), jnp.float32),
            pltpu.VMEM((1120, 512), jnp.bfloat16),
            pltpu.VMEM((420, 512), jnp.bfloat16),
            tuple(pltpu.VMEM((320, 256), jnp.bfloat16) for _ in range(2)),
            pltpu.VMEM((156, 512), jnp.bfloat16),
            tuple(pltpu.VMEM((88, 256), jnp.bfloat16) for _ in range(4)),
        ],
        compiler_params=pltpu.CompilerParams(
            dimension_semantics=("parallel",)),
    )(xf, _pad_permute(), w1im, w2bd, w3bd, b1t, b2t, b3t, s1, s2, s3)

    xf = pooled.reshape(S, 16, 4, 4, 64).transpose(0, 2, 3, 1, 4)
    xf = xf.reshape(B, 1024)

    fw1r = fw1.reshape(4, 4, 128, 64)[:, :, :64, :]
    fw1r = fw1r.reshape(1024, 64).astype(jnp.bfloat16)

    MB = 1024 if B % 1024 == 0 else B
    return pl.pallas_call(
        _mlp_kernel,
        out_shape=jax.ShapeDtypeStruct((B, 10), jnp.float32),
        grid=(B // MB,),
        in_specs=[
            pl.BlockSpec((MB, 1024), lambda s: (s, 0)),
            pl.BlockSpec((1024, 64), lambda s: (0, 0)),
            pl.BlockSpec((1, 64), lambda s: (0, 0)),
            pl.BlockSpec((64, 10), lambda s: (0, 0)),
            pl.BlockSpec((1, 10), lambda s: (0, 0)),
        ],
        out_specs=pl.BlockSpec((MB, 10), lambda s: (s, 0)),
        compiler_params=pltpu.CompilerParams(
            dimension_semantics=("parallel",)),
    )(xf, fw1r, fb1, fw2, fb2)


# speedup vs baseline: 4.3427x; 1.0648x over previous
import numpy as np

import jax
import jax.numpy as jnp
from jax.experimental import pallas as pl
from jax.experimental.pallas import tpu as pltpu

KSIZE = 5
GROUP = 16


def _pool_select(n_rows, wp_in, h_out, w_out, wp_out, pad, k_pad):
    s = np.zeros((n_rows, k_pad), np.float32)
    for ho in range(h_out):
        for wo in range(w_out):
            s[(ho + pad) * wp_out + (wo + pad), 2 * ho * wp_in + 2 * wo] = 1.0
    return jnp.asarray(s, jnp.bfloat16)


def _pad_permute():
    p = np.zeros((1024, 1408), np.float32)
    for h in range(32):
        for w in range(32):
            p[h * 32 + w, (h + 2) * 36 + (w + 2)] = 1.0
    return jnp.asarray(p, jnp.bfloat16)


def _tower_kernel(xf_ref, p_ref, w1_ref, w2_ref, w3_ref,
                  b1_ref, b2_ref, b3_ref,
                  s1_ref, s2_ref, s3_ref, o_ref,
                  xpt_ref, x1_ref, acc1_ref, m1_ref, x2_ref,
                  m2_refs, x3_ref, x3im_refs, m3_refs):
    xpt_ref[...] = jnp.dot(xf_ref[0], p_ref[...],
                           preferred_element_type=jnp.float32
                           ).astype(jnp.bfloat16)
    for t in range(KSIZE * KSIZE):
        kh, kw = divmod(t, KSIZE)
        off = kh * 36 + kw
        x1_ref[48 * t:48 * (t + 1), :] = xpt_ref[:, pl.ds(off, 1152)]
    acc1_ref[...] = jax.lax.dot_general(
        x1_ref[...], w1_ref[...], (((0,), (0,)), ((), ())),
        preferred_element_type=jnp.float32)

    m1 = jnp.maximum(
        jnp.maximum(acc1_ref[pl.ds(0, 1115), :], acc1_ref[pl.ds(1, 1115), :]),
        jnp.maximum(acc1_ref[pl.ds(36, 1115), :], acc1_ref[pl.ds(37, 1115), :]))
    m1_ref[pl.ds(0, 1115), :] = (m1 + b1_ref[...]).astype(jnp.bfloat16)
    m1_ref[pl.ds(1115, 5), :] = jnp.zeros((5, 512), jnp.bfloat16)
    x2_ref[...] = jnp.dot(s1_ref[...], m1_ref[...],
                          preferred_element_type=jnp.float32
                          ).astype(jnp.bfloat16)

    for g in range(2):
        m2_ref = m2_refs[g]
        for si in range(2):
            base = 160 * si
            acc = None
            for t in range(KSIZE * KSIZE):
                kh, kw = divmod(t, KSIZE)
                off = base + kh * 20 + kw
                xs = x2_ref[pl.ds(off, 160), 256 * g:256 * (g + 1)]
                contrib = jnp.dot(xs, w2_ref[t, :, :],
                                  preferred_element_type=jnp.float32)
                acc = contrib if acc is None else acc + contrib
            m2 = jnp.maximum(jnp.maximum(acc[0:139], acc[1:140]),
                             jnp.maximum(acc[20:159], acc[21:160]))
            m2_ref[pl.ds(base, 139), :] = (m2 + b2_ref[...]).astype(jnp.bfloat16)
            m2_ref[pl.ds(base + 139, 21), :] = jnp.zeros((21, 256),
                                                         jnp.bfloat16)
        x3_ref[:, 256 * g:256 * (g + 1)] = jnp.dot(
            s2_ref[...], m2_ref[...],
            preferred_element_type=jnp.float32).astype(jnp.bfloat16)

    for g in range(4):
        m3_ref = m3_refs[g]
        x3im_ref = x3im_refs[g % 2]
        for t in range(KSIZE * KSIZE):
            kh, kw = divmod(t, KSIZE)
            off = kh * 12 + kw
            x3im_ref[:, 128 * t:128 * (t + 1)] = \
                x3_ref[pl.ds(off, 96), 128 * g:128 * (g + 1)]
        acc = jnp.dot(x3im_ref[...], w3_ref[...],
                      preferred_element_type=jnp.float32)
        m3 = jnp.maximum(jnp.maximum(acc[0:83], acc[1:84]),
                         jnp.maximum(acc[12:95], acc[13:96]))
        m3_ref[pl.ds(0, 83), :] = (m3 + b3_ref[...]).astype(jnp.bfloat16)
        m3_ref[pl.ds(83, 5), :] = jnp.zeros((5, 256), jnp.bfloat16)
        o_ref[0, :, 256 * g:256 * (g + 1)] = jnp.dot(
            s3_ref[...], m3_ref[...],
            preferred_element_type=jnp.float32).astype(jnp.bfloat16)


def _mlp_kernel(x_ref, w1_ref, b1_ref, w2_ref, b2_ref, o_ref):
    h = jnp.dot(x_ref[...], w1_ref[...],
                preferred_element_type=jnp.float32) + b1_ref[...]
    o_ref[...] = jnp.dot(h, w2_ref[...],
                         preferred_element_type=jnp.float32) + b2_ref[...]


@jax.jit
def kernel(w1, b1, w2, b2, w3, b3, fw1, fb1, fw2, fb2, x):
    B = x.shape[0]
    S = B // GROUP

    xf = x.astype(jnp.bfloat16).reshape(S, GROUP * 3, 1024)

    eye16 = jnp.eye(16, dtype=w1.dtype)
    eye8 = jnp.eye(8, dtype=w1.dtype)
    eye4 = jnp.eye(4, dtype=w1.dtype)
    w1bd = jnp.einsum('ab,tkc->takbc', eye16, w1[:, :3, :32])
    w1im = w1bd.reshape(25 * 48, 512).astype(jnp.bfloat16)
    w2bd = jnp.einsum('ab,tkc->takbc', eye8, w2[:, :32, :32])
    w2bd = w2bd.reshape(25, 256, 256).astype(jnp.bfloat16)
    w3bd = jnp.einsum('ab,tkc->takbc', eye4, w3[:, :32, :64])
    w3bd = w3bd.reshape(25 * 128, 256).astype(jnp.bfloat16)
    b1t = jnp.tile(b1[:, :32], (1, 16))
    b2t = jnp.tile(b2[:, :32], (1, 8))
    b3t = jnp.tile(b3[:, :64], (1, 4))

    s1 = _pool_select(420, 36, 16, 16, 20, 2, 1120)
    s2 = _pool_select(156, 20, 8, 8, 12, 2, 320)
    s3 = _pool_select(16, 12, 4, 4, 4, 0, 88)

    pooled = pl.pallas_call(
        _tower_kernel,
        out_shape=jax.ShapeDtypeStruct((S, 16, 1024), jnp.bfloat16),
        grid=(S,),
        in_specs=[
            pl.BlockSpec((1, GROUP * 3, 1024), lambda s: (s, 0, 0)),
            pl.BlockSpec((1024, 1408), lambda s: (0, 0)),
            pl.BlockSpec((25 * 48, 512), lambda s: (0, 0)),
            pl.BlockSpec((25, 256, 256), lambda s: (0, 0, 0)),
            pl.BlockSpec((25 * 128, 256), lambda s: (0, 0)),
            pl.BlockSpec((1, 512), lambda s: (0, 0)),
            pl.BlockSpec((1, 256), lambda s: (0, 0)),
            pl.BlockSpec((1, 256), lambda s: (0, 0)),
            pl.BlockSpec((420, 1120), lambda s: (0, 0)),
            pl.BlockSpec((156, 320), lambda s: (0, 0)),
            pl.BlockSpec((16, 88), lambda s: (0, 0)),
        ],
        out_specs=pl.BlockSpec((1, 16, 1024), lambda s: (s, 0, 0)),
        scratch_shapes=[
            pltpu.VMEM((48, 1408), jnp.bfloat16),
            pltpu.VMEM((1200, 1152), jnp.bfloat16),
            pltpu.VMEM((1152, 512), jnp.float32),
            pltpu.VMEM((1120, 512), jnp.bfloat16),
            pltpu.VMEM((420, 512), jnp.bfloat16),
            tuple(pltpu.VMEM((320, 256), jnp.bfloat16) for _ in range(2)),
            pltpu.VMEM((156, 512), jnp.bfloat16),
            tuple(pltpu.VMEM((96, 3200), jnp.bfloat16) for _ in range(2)),
            tuple(pltpu.VMEM((88, 256), jnp.bfloat16) for _ in range(4)),
        ],
        compiler_params=pltpu.CompilerParams(
            dimension_semantics=("parallel",)),
    )(xf, _pad_permute(), w1im, w2bd, w3bd, b1t, b2t, b3t, s1, s2, s3)

    xf = pooled.reshape(S, 16, 4, 4, 64).transpose(0, 2, 3, 1, 4)
    xf = xf.reshape(B, 1024)

    fw1r = fw1.reshape(4, 4, 128, 64)[:, :, :64, :]
    fw1r = fw1r.reshape(1024, 64).astype(jnp.bfloat16)

    MB = 1024 if B % 1024 == 0 else B
    return pl.pallas_call(
        _mlp_kernel,
        out_shape=jax.ShapeDtypeStruct((B, 10), jnp.float32),
        grid=(B // MB,),
        in_specs=[
            pl.BlockSpec((MB, 1024), lambda s: (s, 0)),
            pl.BlockSpec((1024, 64), lambda s: (0, 0)),
            pl.BlockSpec((1, 64), lambda s: (0, 0)),
            pl.BlockSpec((64, 10), lambda s: (0, 0)),
            pl.BlockSpec((1, 10), lambda s: (0, 0)),
        ],
        out_specs=pl.BlockSpec((MB, 10), lambda s: (s, 0)),
        compiler_params=pltpu.CompilerParams(
            dimension_semantics=("parallel",)),
    )(xf, fw1r, fb1, fw2, fb2)


# speedup vs baseline: 4.3859x; 1.0100x over previous
import numpy as np

import jax
import jax.numpy as jnp
from jax.experimental import pallas as pl
from jax.experimental.pallas import tpu as pltpu

KSIZE = 5
GROUP = 16


def _pool_select(n_rows, wp_in, h_out, w_out, wp_out, pad, k_pad):
    s = np.zeros((n_rows, k_pad), np.float32)
    for ho in range(h_out):
        for wo in range(w_out):
            s[(ho + pad) * wp_out + (wo + pad), 2 * ho * wp_in + 2 * wo] = 1.0
    return jnp.asarray(s, jnp.bfloat16)


def _pad_permute():
    p = np.zeros((1024, 1408), np.float32)
    for h in range(32):
        for w in range(32):
            p[h * 32 + w, (h + 2) * 36 + (w + 2)] = 1.0
    return jnp.asarray(p, jnp.bfloat16)


def _tower_kernel(xf_ref, p_ref, w1_ref, w2_ref, w3_ref,
                  b1_ref, b2_ref, b3_ref,
                  s1_ref, s2_ref, s3_ref, o_ref,
                  xpt_ref, x1_ref, acc1_ref, m1_ref, x2_ref,
                  m2_refs, x3_ref, x3im_refs, m3_refs):
    xpt_ref[...] = jnp.dot(xf_ref[0].astype(jnp.bfloat16), p_ref[...],
                           preferred_element_type=jnp.float32
                           ).astype(jnp.bfloat16)
    for t in range(KSIZE * KSIZE):
        kh, kw = divmod(t, KSIZE)
        off = kh * 36 + kw
        x1_ref[48 * t:48 * (t + 1), :] = xpt_ref[:, pl.ds(off, 1152)]
    x1_ref[1200:1216, :] = jnp.ones((16, 1152), jnp.bfloat16)
    acc1_ref[...] = jax.lax.dot_general(
        x1_ref[...], w1_ref[...], (((0,), (0,)), ((), ())),
        preferred_element_type=jnp.float32)

    m1 = jnp.maximum(
        jnp.maximum(acc1_ref[pl.ds(0, 1115), :], acc1_ref[pl.ds(1, 1115), :]),
        jnp.maximum(acc1_ref[pl.ds(36, 1115), :], acc1_ref[pl.ds(37, 1115), :]))
    m1_ref[pl.ds(0, 1115), :] = m1.astype(jnp.bfloat16)
    m1_ref[pl.ds(1115, 5), :] = jnp.zeros((5, 512), jnp.bfloat16)
    x2_ref[...] = jnp.dot(s1_ref[...], m1_ref[...],
                          preferred_element_type=jnp.float32
                          ).astype(jnp.bfloat16)

    for g in range(2):
        m2_ref = m2_refs[g]
        for si in range(2):
            base = 160 * si
            acc = None
            for t in range(KSIZE * KSIZE):
                kh, kw = divmod(t, KSIZE)
                off = base + kh * 20 + kw
                xs = x2_ref[pl.ds(off, 160), 256 * g:256 * (g + 1)]
                contrib = jnp.dot(xs, w2_ref[t, :, :],
                                  preferred_element_type=jnp.float32)
                acc = contrib if acc is None else acc + contrib
            m2 = jnp.maximum(jnp.maximum(acc[0:139], acc[1:140]),
                             jnp.maximum(acc[20:159], acc[21:160]))
            m2_ref[pl.ds(base, 139), :] = (m2 + b2_ref[...]).astype(jnp.bfloat16)
            m2_ref[pl.ds(base + 139, 21), :] = jnp.zeros((21, 256),
                                                         jnp.bfloat16)
        x3_ref[:, 256 * g:256 * (g + 1)] = jnp.dot(
            s2_ref[...], m2_ref[...],
            preferred_element_type=jnp.float32).astype(jnp.bfloat16)

    for g in range(4):
        m3_ref = m3_refs[g]
        x3im_ref = x3im_refs[g % 2]
        for t in range(KSIZE * KSIZE):
            kh, kw = divmod(t, KSIZE)
            off = kh * 12 + kw
            x3im_ref[:, 128 * t:128 * (t + 1)] = \
                x3_ref[pl.ds(off, 96), 128 * g:128 * (g + 1)]
        acc = jnp.dot(x3im_ref[...], w3_ref[...],
                      preferred_element_type=jnp.float32)
        m3 = jnp.maximum(jnp.maximum(acc[0:83], acc[1:84]),
                         jnp.maximum(acc[12:95], acc[13:96]))
        m3_ref[pl.ds(0, 83), :] = (m3 + b3_ref[...]).astype(jnp.bfloat16)
        m3_ref[pl.ds(83, 5), :] = jnp.zeros((5, 256), jnp.bfloat16)
        o_ref[0, :, 256 * g:256 * (g + 1)] = jnp.dot(
            s3_ref[...], m3_ref[...],
            preferred_element_type=jnp.float32).astype(jnp.bfloat16)


def _mlp_kernel(x_ref, w1_ref, b1_ref, w2_ref, b2_ref, o_ref):
    h = jnp.dot(x_ref[...], w1_ref[...],
                preferred_element_type=jnp.float32) + b1_ref[...]
    o_ref[...] = jnp.dot(h, w2_ref[...],
                         preferred_element_type=jnp.float32) + b2_ref[...]


@jax.jit
def kernel(w1, b1, w2, b2, w3, b3, fw1, fb1, fw2, fb2, x):
    B = x.shape[0]
    S = B // GROUP

    xf = x.reshape(S, GROUP * 3, 1024)

    eye16 = jnp.eye(16, dtype=w1.dtype)
    eye8 = jnp.eye(8, dtype=w1.dtype)
    eye4 = jnp.eye(4, dtype=w1.dtype)
    w1bd = jnp.einsum('ab,tkc->takbc', eye16, w1[:, :3, :32])
    w1im = w1bd.reshape(25 * 48, 512)
    b1bd = jnp.einsum('ab,c->abc', eye16, b1[0, :32]).reshape(16, 512)
    w1im = jnp.concatenate([w1im, b1bd], axis=0).astype(jnp.bfloat16)
    w2bd = jnp.einsum('ab,tkc->takbc', eye8, w2[:, :32, :32])
    w2bd = w2bd.reshape(25, 256, 256).astype(jnp.bfloat16)
    w3bd = jnp.einsum('ab,tkc->takbc', eye4, w3[:, :32, :64])
    w3bd = w3bd.reshape(25 * 128, 256).astype(jnp.bfloat16)
    b1t = jnp.tile(b1[:, :32], (1, 16))
    b2t = jnp.tile(b2[:, :32], (1, 8))
    b3t = jnp.tile(b3[:, :64], (1, 4))

    s1 = _pool_select(420, 36, 16, 16, 20, 2, 1120)
    s2 = _pool_select(156, 20, 8, 8, 12, 2, 320)
    s3 = _pool_select(16, 12, 4, 4, 4, 0, 88)

    pooled = pl.pallas_call(
        _tower_kernel,
        out_shape=jax.ShapeDtypeStruct((S, 16, 1024), jnp.bfloat16),
        grid=(S,),
        in_specs=[
            pl.BlockSpec((1, GROUP * 3, 1024), lambda s: (s, 0, 0)),
            pl.BlockSpec((1024, 1408), lambda s: (0, 0)),
            pl.BlockSpec((25 * 48 + 16, 512), lambda s: (0, 0)),
            pl.BlockSpec((25, 256, 256), lambda s: (0, 0, 0)),
            pl.BlockSpec((25 * 128, 256), lambda s: (0, 0)),
            pl.BlockSpec((1, 512), lambda s: (0, 0)),
            pl.BlockSpec((1, 256), lambda s: (0, 0)),
            pl.BlockSpec((1, 256), lambda s: (0, 0)),
            pl.BlockSpec((420, 1120), lambda s: (0, 0)),
            pl.BlockSpec((156, 320), lambda s: (0, 0)),
            pl.BlockSpec((16, 88), lambda s: (0, 0)),
        ],
        out_specs=pl.BlockSpec((1, 16, 1024), lambda s: (s, 0, 0)),
        scratch_shapes=[
            pltpu.VMEM((48, 1408), jnp.bfloat16),
            pltpu.VMEM((1216, 1152), jnp.bfloat16),
            pltpu.VMEM((1152, 512), jnp.float32),
            pltpu.VMEM((1120, 512), jnp.bfloat16),
            pltpu.VMEM((420, 512), jnp.bfloat16),
            tuple(pltpu.VMEM((320, 256), jnp.bfloat16) for _ in range(2)),
            pltpu.VMEM((156, 512), jnp.bfloat16),
            tuple(pltpu.VMEM((96, 3200), jnp.bfloat16) for _ in range(2)),
            tuple(pltpu.VMEM((88, 256), jnp.bfloat16) for _ in range(4)),
        ],
        compiler_params=pltpu.CompilerParams(
            dimension_semantics=("parallel",)),
    )(xf, _pad_permute(), w1im, w2bd, w3bd, b1t, b2t, b3t, s1, s2, s3)

    xf = pooled.reshape(S, 16, 4, 4, 64).transpose(0, 2, 3, 1, 4)
    xf = xf.reshape(B, 1024)

    fw1r = fw1.reshape(4, 4, 128, 64)[:, :, :64, :]
    fw1r = fw1r.reshape(1024, 64).astype(jnp.bfloat16)

    MB = 1024 if B % 1024 == 0 else B
    return pl.pallas_call(
        _mlp_kernel,
        out_shape=jax.ShapeDtypeStruct((B, 10), jnp.float32),
        grid=(B // MB,),
        in_specs=[
            pl.BlockSpec((MB, 1024), lambda s: (s, 0)),
            pl.BlockSpec((1024, 64), lambda s: (0, 0)),
            pl.BlockSpec((1, 64), lambda s: (0, 0)),
            pl.BlockSpec((64, 10), lambda s: (0, 0)),
            pl.BlockSpec((1, 10), lambda s: (0, 0)),
        ],
        out_specs=pl.BlockSpec((MB, 10), lambda s: (s, 0)),
        compiler_params=pltpu.CompilerParams(
            dimension_semantics=("parallel",)),
    )(xf, fw1r, fb1, fw2, fb2)


# speedup vs baseline: 4.3896x; 1.0008x over previous
import numpy as np

import jax
import jax.numpy as jnp
from jax.experimental import pallas as pl
from jax.experimental.pallas import tpu as pltpu

KSIZE = 5
GROUP = 16


def _pool_select(n_rows, wp_in, h_out, w_out, wp_out, pad, k_pad):
    s = np.zeros((n_rows, k_pad), np.float32)
    for ho in range(h_out):
        for wo in range(w_out):
            s[(ho + pad) * wp_out + (wo + pad), 2 * ho * wp_in + 2 * wo] = 1.0
    return jnp.asarray(s, jnp.bfloat16)


def _pad_permute():
    p = np.zeros((1024, 1408), np.float32)
    for h in range(32):
        for w in range(32):
            p[h * 32 + w, (h + 2) * 36 + (w + 2)] = 1.0
    return jnp.asarray(p, jnp.bfloat16)


def _tower_kernel(xf_ref, p_ref, w1_ref, w2_ref, w3_ref,
                  b2_ref, b3_ref,
                  s1_ref, s2_ref, s3_ref, o_ref,
                  xpt_ref, x1_ref, acc1_ref, m1_ref, x2_ref,
                  m2_refs, x3_ref, x3im_refs, m3_refs):
    xpt_ref[...] = jnp.dot(xf_ref[0].astype(jnp.bfloat16), p_ref[...],
                           preferred_element_type=jnp.float32
                           ).astype(jnp.bfloat16)
    for t in range(KSIZE * KSIZE):
        kh, kw = divmod(t, KSIZE)
        off = kh * 36 + kw
        x1_ref[48 * t:48 * (t + 1), :] = xpt_ref[:, pl.ds(off, 1152)]
    x1_ref[1200:1216, :] = jnp.ones((16, 1152), jnp.bfloat16)
    acc1_ref[...] = jax.lax.dot_general(
        x1_ref[...], w1_ref[...], (((0,), (0,)), ((), ())),
        preferred_element_type=jnp.float32)

    m1 = jnp.maximum(
        jnp.maximum(acc1_ref[pl.ds(0, 1115), :], acc1_ref[pl.ds(1, 1115), :]),
        jnp.maximum(acc1_ref[pl.ds(36, 1115), :], acc1_ref[pl.ds(37, 1115), :]))
    m1_ref[pl.ds(0, 1115), :] = m1.astype(jnp.bfloat16)
    m1_ref[pl.ds(1115, 5), :] = jnp.zeros((5, 512), jnp.bfloat16)
    x2_ref[...] = jnp.dot(s1_ref[...], m1_ref[...],
                          preferred_element_type=jnp.float32
                          ).astype(jnp.bfloat16)

    for g in range(2):
        m2_ref = m2_refs[g]
        for si in range(2):
            base = 160 * si
            acc = None
            for t in range(KSIZE * KSIZE):
                kh, kw = divmod(t, KSIZE)
                off = base + kh * 20 + kw
                xs = x2_ref[pl.ds(off, 160), 256 * g:256 * (g + 1)]
                contrib = jnp.dot(xs, w2_ref[t, :, :],
                                  preferred_element_type=jnp.float32)
                acc = contrib if acc is None else acc + contrib
            m2 = jnp.maximum(jnp.maximum(acc[0:139], acc[1:140]),
                             jnp.maximum(acc[20:159], acc[21:160]))
            m2_ref[pl.ds(base, 139), :] = (m2 + b2_ref[...]).astype(jnp.bfloat16)
            m2_ref[pl.ds(base + 139, 21), :] = jnp.zeros((21, 256),
                                                         jnp.bfloat16)
        x3_ref[:, 256 * g:256 * (g + 1)] = jnp.dot(
            s2_ref[...], m2_ref[...],
            preferred_element_type=jnp.float32).astype(jnp.bfloat16)

    for g in range(4):
        m3_ref = m3_refs[g]
        x3im_ref = x3im_refs[g % 2]
        for t in range(KSIZE * KSIZE):
            kh, kw = divmod(t, KSIZE)
            off = kh * 12 + kw
            x3im_ref[:, 128 * t:128 * (t + 1)] = \
                x3_ref[pl.ds(off, 96), 128 * g:128 * (g + 1)]
        acc = jnp.dot(x3im_ref[...], w3_ref[...],
                      preferred_element_type=jnp.float32)
        m3 = jnp.maximum(jnp.maximum(acc[0:83], acc[1:84]),
                         jnp.maximum(acc[12:95], acc[13:96]))
        m3_ref[pl.ds(0, 83), :] = (m3 + b3_ref[...]).astype(jnp.bfloat16)
        m3_ref[pl.ds(83, 5), :] = jnp.zeros((5, 256), jnp.bfloat16)
        o_ref[0, :, 256 * g:256 * (g + 1)] = jnp.dot(
            s3_ref[...], m3_ref[...],
            preferred_element_type=jnp.float32).astype(jnp.bfloat16)


def _mlp_kernel(x_ref, w1_ref, b1_ref, w2_ref, b2_ref, o_ref):
    h = jnp.dot(x_ref[...], w1_ref[...],
                preferred_element_type=jnp.float32) + b1_ref[...]
    o_ref[...] = jnp.dot(h, w2_ref[...],
                         preferred_element_type=jnp.float32) + b2_ref[...]


@jax.jit
def kernel(w1, b1, w2, b2, w3, b3, fw1, fb1, fw2, fb2, x):
    B = x.shape[0]
    S = B // GROUP

    xf = x.reshape(S, GROUP * 3, 1024)

    eye16 = jnp.eye(16, dtype=w1.dtype)
    eye8 = jnp.eye(8, dtype=w1.dtype)
    eye4 = jnp.eye(4, dtype=w1.dtype)
    w1bd = jnp.einsum('ab,tkc->takbc', eye16, w1[:, :3, :32])
    w1im = w1bd.reshape(25 * 48, 512)
    b1bd = jnp.einsum('ab,c->abc', eye16, b1[0, :32]).reshape(16, 512)
    w1im = jnp.concatenate([w1im, b1bd], axis=0).astype(jnp.bfloat16)
    w2bd = jnp.einsum('ab,tkc->takbc', eye8, w2[:, :32, :32])
    w2bd = w2bd.reshape(25, 256, 256).astype(jnp.bfloat16)
    w3bd = jnp.einsum('ab,tkc->takbc', eye4, w3[:, :32, :64])
    w3bd = w3bd.reshape(25 * 128, 256).astype(jnp.bfloat16)
    b2t = jnp.tile(b2[:, :32], (1, 8))
    b3t = jnp.tile(b3[:, :64], (1, 4))

    s1 = _pool_select(420, 36, 16, 16, 20, 2, 1120)
    s2 = _pool_select(156, 20, 8, 8, 12, 2, 320)
    s3 = _pool_select(16, 12, 4, 4, 4, 0, 88)

    pooled = pl.pallas_call(
        _tower_kernel,
        out_shape=jax.ShapeDtypeStruct((S, 16, 1024), jnp.bfloat16),
        grid=(S,),
        in_specs=[
            pl.BlockSpec((1, GROUP * 3, 1024), lambda s: (s, 0, 0)),
            pl.BlockSpec((1024, 1408), lambda s: (0, 0)),
            pl.BlockSpec((25 * 48 + 16, 512), lambda s: (0, 0)),
            pl.BlockSpec((25, 256, 256), lambda s: (0, 0, 0)),
            pl.BlockSpec((25 * 128, 256), lambda s: (0, 0)),
            pl.BlockSpec((1, 256), lambda s: (0, 0)),
            pl.BlockSpec((1, 256), lambda s: (0, 0)),
            pl.BlockSpec((420, 1120), lambda s: (0, 0)),
            pl.BlockSpec((156, 320), lambda s: (0, 0)),
            pl.BlockSpec((16, 88), lambda s: (0, 0)),
        ],
        out_specs=pl.BlockSpec((1, 16, 1024), lambda s: (s, 0, 0)),
        scratch_shapes=[
            pltpu.VMEM((48, 1408), jnp.bfloat16),
            pltpu.VMEM((1216, 1152), jnp.bfloat16),
            pltpu.VMEM((1152, 512), jnp.float32),
            pltpu.VMEM((1120, 512), jnp.bfloat16),
            pltpu.VMEM((420, 512), jnp.bfloat16),
            tuple(pltpu.VMEM((320, 256), jnp.bfloat16) for _ in range(2)),
            pltpu.VMEM((156, 512), jnp.bfloat16),
            tuple(pltpu.VMEM((96, 3200), jnp.bfloat16) for _ in range(2)),
            tuple(pltpu.VMEM((88, 256), jnp.bfloat16) for _ in range(4)),
        ],
        compiler_params=pltpu.CompilerParams(
            dimension_semantics=("parallel",)),
    )(xf, _pad_permute(), w1im, w2bd, w3bd, b2t, b3t, s1, s2, s3)

    xf = pooled.reshape(S, 16, 4, 4, 64).transpose(0, 2, 3, 1, 4)
    xf = xf.reshape(B, 1024)

    fw1r = fw1.reshape(4, 4, 128, 64)[:, :, :64, :]
    fw1r = fw1r.reshape(1024, 64).astype(jnp.bfloat16)

    MB = 1024 if B % 1024 == 0 else B
    return pl.pallas_call(
        _mlp_kernel,
        out_shape=jax.ShapeDtypeStruct((B, 10), jnp.float32),
        grid=(B // MB,),
        in_specs=[
            pl.BlockSpec((MB, 1024), lambda s: (s, 0)),
            pl.BlockSpec((1024, 64), lambda s: (0, 0)),
            pl.BlockSpec((1, 64), lambda s: (0, 0)),
            pl.BlockSpec((64, 10), lambda s: (0, 0)),
            pl.BlockSpec((1, 10), lambda s: (0, 0)),
        ],
        out_specs=pl.BlockSpec((MB, 10), lambda s: (s, 0)),
        compiler_params=pltpu.CompilerParams(
            dimension_semantics=("parallel",)),
    )(xf, fw1r, fb1, fw2, fb2)
